# baseline (speedup 1.0000x reference)
import functools

import jax
import jax.numpy as jnp
from jax import lax
from jax.experimental import pallas as pl
from jax.experimental.pallas import tpu as pltpu
from jax.experimental.pallas import tpu_sc as plsc

H = 128
CH = 128
NW = 32
NP = 10240
BLK_E = 1280
BLK_N = 2000


def _sc_mesh():
    return plsc.VectorSubcoreMesh(core_axis_name="c", subcore_axis_name="s")


def _rne16(v):
    bits = jax.lax.bitcast_convert_type(v, jnp.uint32)
    return (bits + jnp.uint32(0x7FFF) + ((bits >> 16) & jnp.uint32(1))) >> 16


def _pack_pair(lo, hi):
    return jax.lax.bitcast_convert_type(_rne16(lo) | (_rne16(hi) << 16), jnp.int32)


def _unpack2(g):
    u = jax.lax.bitcast_convert_type(g, jnp.uint32)
    lo = jax.lax.bitcast_convert_type(u << 16, jnp.float32)
    hi = jax.lax.bitcast_convert_type(u & jnp.uint32(0xFFFF0000), jnp.float32)
    return lo, hi


def _gather_pair(A, B, row, col):
    E = row.shape[0]
    D = A.shape[1]
    dt = A.dtype
    nch = E // CH
    per_w = (nch + NW - 1) // NW
    rpc = CH * D // 128

    @functools.partial(
        pl.kernel,
        mesh=_sc_mesh(),
        compiler_params=pltpu.CompilerParams(use_tc_tiling_on_sc=(D % 128 == 0)),
        out_type=[jax.ShapeDtypeStruct((E, D), dt),
                  jax.ShapeDtypeStruct((E, D), dt)],
        scratch_types=[pltpu.VMEM((1, CH), jnp.int32),
                       pltpu.VMEM((1, CH), jnp.int32),
                       pltpu.VMEM((CH, D), dt),
                       pltpu.VMEM((CH, D), dt),
                       pltpu.SemaphoreType.DMA,
                       pltpu.SemaphoreType.DMA],
    )
    def k(a_hbm, b_hbm, row_hbm, col_hbm, ga_hbm, gb_hbm, ri, ci, ba, bb, s1, s2):
        wid = lax.axis_index("s") * 2 + lax.axis_index("c")

        @pl.loop(0, per_w)
        def _(i):
            c = wid + i * NW

            @pl.when(c < nch)
            def _():
                base = c * CH
                pltpu.sync_copy(row_hbm.at[pl.ds(base, CH)], ri.at[0])
                pltpu.sync_copy(col_hbm.at[pl.ds(base, CH)], ci.at[0])
                cp1 = pltpu.async_copy(a_hbm.at[ri.at[0]], ba, s1)
                cp2 = pltpu.async_copy(b_hbm.at[ci.at[0]], bb, s2)
                cp1.wait()
                cp2.wait()
                pltpu.sync_copy(ba, ga_hbm.at[pl.ds(base, CH)])
                pltpu.sync_copy(bb, gb_hbm.at[pl.ds(base, CH)])

    return k(A, B, row, col)


def _gather_fused(A, B, row, col):
    E = row.shape[0]
    D = A.shape[1]
    nch = E // CH
    per_w = (nch + NW - 1) // NW

    @functools.partial(
        pl.kernel,
        mesh=_sc_mesh(),
        compiler_params=pltpu.CompilerParams(use_tc_tiling_on_sc=False),
        out_type=jax.ShapeDtypeStruct((E, 2 * D), jnp.int32),
        scratch_types=[pltpu.VMEM((1, CH), jnp.int32),
                       pltpu.VMEM((1, CH), jnp.int32),
                       pltpu.VMEM((CH, D), jnp.int32),
                       pltpu.VMEM((CH, D), jnp.int32),
                       pltpu.VMEM((CH, 2 * D), jnp.int32),
                       pltpu.SemaphoreType.DMA,
                       pltpu.SemaphoreType.DMA],
    )
    def k(a_hbm, b_hbm, row_hbm, col_hbm, g_hbm, ri, ci, ba, bb, buf, s1, s2):
        wid = lax.axis_index("s") * 2 + lax.axis_index("c")

        @pl.loop(0, per_w)
        def _(i):
            c = wid + i * NW

            @pl.when(c < nch)
            def _():
                base = c * CH
                pltpu.sync_copy(row_hbm.at[pl.ds(base, CH)], ri.at[0])
                pltpu.sync_copy(col_hbm.at[pl.ds(base, CH)], ci.at[0])
                cp1 = pltpu.async_copy(a_hbm.at[ri.at[0]], ba, s1)
                cp2 = pltpu.async_copy(b_hbm.at[ci.at[0]], bb, s2)
                cp1.wait()
                cp2.wait()

                @pl.loop(0, CH)
                def _(r):
                    for j in range(D // 16):
                        buf[r, pl.ds(16 * j, 16)] = ba[r, pl.ds(16 * j, 16)]
                        buf[r, pl.ds(D + 16 * j, 16)] = bb[r, pl.ds(16 * j, 16)]

                pltpu.sync_copy(buf, g_hbm.at[pl.ds(base, CH)])

    return k(A, B, row, col)


def _sc_scatter_add(F, row, zrows):
    E, D = F.shape
    nch = E // CH
    per_w = (nch + NW - 1) // NW
    rpt = NP // 16

    @functools.partial(
        pl.kernel,
        mesh=_sc_mesh(),
        compiler_params=pltpu.CompilerParams(use_tc_tiling_on_sc=(D % 128 == 0)),
        out_type=jax.ShapeDtypeStruct((2, NP, D), jnp.float32),
        scratch_types=[pltpu.VMEM((1, CH), jnp.int32),
                       pltpu.VMEM((CH, D), jnp.float32),
                       pltpu.VMEM_SHARED((NP, D), jnp.float32)],
    )
    def k(f_hbm, row_hbm, z_hbm, p_hbm, ri, fb, acc):
        cid = lax.axis_index("c")
        sid = lax.axis_index("s")
        wid = sid * 2 + cid

        @pl.loop(0, rpt // CH)
        def _(j):
            pltpu.sync_copy(z_hbm, acc.at[pl.ds(sid * rpt + j * CH, CH)])

        plsc.subcore_barrier()

        @pl.loop(0, per_w)
        def _(i):
            c = wid + i * NW

            @pl.when(c < nch)
            def _():
                base = c * CH
                pltpu.sync_copy(row_hbm.at[pl.ds(base, CH)], ri.at[0])
                pltpu.sync_copy(f_hbm.at[pl.ds(base, CH)], fb)
                pltpu.sync_copy(fb, acc.at[ri.at[0]], add=True)

        plsc.subcore_barrier()

        @pl.loop(0, rpt // CH)
        def _(j):
            off = sid * rpt + j * CH
            pltpu.sync_copy(acc.at[pl.ds(off, CH)], p_hbm.at[cid, pl.ds(off, CH)])

    return k(F, row, zrows)


def _prep(h, W1a, W1b, b1):
    N = h.shape[0]

    def body(h_ref, wa_ref, wb_ref, b_ref, a_ref, bo_ref):
        hv = h_ref[...]
        wav = wa_ref[...]
        wbv = wb_ref[...]
        bv = b_ref[...]
        a_ref[...] = _pack_pair(
            jnp.dot(hv, wav[:, :64], preferred_element_type=jnp.float32) + bv[:, :64],
            jnp.dot(hv, wav[:, 64:], preferred_element_type=jnp.float32) + bv[:, 64:])
        bo_ref[...] = _pack_pair(
            jnp.dot(hv, wbv[:, :64], preferred_element_type=jnp.float32),
            jnp.dot(hv, wbv[:, 64:], preferred_element_type=jnp.float32))

    return pl.pallas_call(
        body,
        grid=(N // BLK_N,),
        in_specs=[pl.BlockSpec((BLK_N, H), lambda i: (i, 0)),
                  pl.BlockSpec((H, H), lambda i: (0, 0)),
                  pl.BlockSpec((H, H), lambda i: (0, 0)),
                  pl.BlockSpec((1, H), lambda i: (0, 0))],
        out_specs=[pl.BlockSpec((BLK_N, H // 2), lambda i: (i, 0)),
                   pl.BlockSpec((BLK_N, H // 2), lambda i: (i, 0))],
        out_shape=[jax.ShapeDtypeStruct((N, H // 2), jnp.int32),
                   jax.ShapeDtypeStruct((N, H // 2), jnp.int32)],
    )(h, W1a, W1b, b1.reshape(1, H))


def _dprep(XR8, XC8, EAP, SG):
    R = XR8.shape[0]
    BR = BLK_E // 8

    def body(xr_ref, xc_ref, ea_ref, sg_ref, d_ref):
        diff = xr_ref[...] - xc_ref[...]
        radial = jnp.dot(diff * diff, sg_ref[...], preferred_element_type=jnp.float32)
        norm = jnp.sqrt(radial + 1e-8)
        cd = diff / (norm + 1.0)
        lane = lax.broadcasted_iota(jnp.int32, diff.shape, 1) % 16
        d_ref[...] = jnp.where(lane < 3, cd,
                               jnp.where(lane == 3, radial, 0.0)) + ea_ref[...]

    return pl.pallas_call(
        body,
        grid=(R // BR,),
        in_specs=[pl.BlockSpec((BR, 128), lambda i: (i, 0)),
                  pl.BlockSpec((BR, 128), lambda i: (i, 0)),
                  pl.BlockSpec((BR, 128), lambda i: (i, 0)),
                  pl.BlockSpec((128, 128), lambda i: (0, 0))],
        out_specs=pl.BlockSpec((BR, 128), lambda i: (i, 0)),
        out_shape=jax.ShapeDtypeStruct((R, 128), jnp.float32),
    )(XR8, XC8, EAP, SG)


def _edge_gcl(G, Dm, W2, b2, wa, ba, wr, we):
    E = G.shape[0]

    def body(g_ref, d_ref, w2_ref, b2_ref, wa_ref, ba_ref, wr_ref, we_ref, f_ref):
        d = d_ref[...]
        first, second = _unpack2(g_ref[...])
        rad = d[:, 3:4]
        ea = d[:, 4:5]
        wrv = wr_ref[...]
        wev = we_ref[...]
        t_lo = jax.nn.silu(first[:, :64] + first[:, 64:]
                           + rad * wrv[:, :64] + ea * wev[:, :64])
        t_hi = jax.nn.silu(second[:, :64] + second[:, 64:]
                           + rad * wrv[:, 64:] + ea * wev[:, 64:])
        w2v = w2_ref[...]
        mij = jax.nn.silu(jnp.dot(t_lo.astype(jnp.bfloat16), w2v[:64],
                                  preferred_element_type=jnp.float32)
                          + jnp.dot(t_hi.astype(jnp.bfloat16), w2v[64:],
                                    preferred_element_type=jnp.float32)
                          + b2_ref[...])
        att = jax.nn.sigmoid(jnp.sum(mij * wa_ref[...], axis=1, keepdims=True) + ba_ref[:, 0:1])
        f_ref[...] = mij * att

    return pl.pallas_call(
        body,
        grid=(E // BLK_E,),
        in_specs=[pl.BlockSpec((BLK_E, H), lambda i: (i, 0)),
                  pl.BlockSpec((BLK_E, 16), lambda i: (i, 0)),
                  pl.BlockSpec((H, H), lambda i: (0, 0)),
                  pl.BlockSpec((1, H), lambda i: (0, 0)),
                  pl.BlockSpec((1, H), lambda i: (0, 0)),
                  pl.BlockSpec((1, H), lambda i: (0, 0)),
                  pl.BlockSpec((1, H), lambda i: (0, 0)),
                  pl.BlockSpec((1, H), lambda i: (0, 0))],
        out_specs=pl.BlockSpec((BLK_E, H), lambda i: (i, 0)),
        out_shape=jax.ShapeDtypeStruct((E, H), jnp.float32),
    )(G, Dm, W2.astype(jnp.bfloat16), b2.reshape(1, H), wa.reshape(1, H),
      jnp.broadcast_to(ba.reshape(1, 1), (1, H)), wr.reshape(1, H), we.reshape(1, H))


def _edge_coord(G, Dm, W2, b2, w3, wr, we):
    E = G.shape[0]

    def body(g_ref, d_ref, w2_ref, b2_ref, w3_ref, wr_ref, we_ref, t_ref):
        d = d_ref[...]
        first, second = _unpack2(g_ref[...])
        rad = d[:, 3:4]
        ea = d[:, 4:5]
        wrv = wr_ref[...]
        wev = we_ref[...]
        t_lo = jax.nn.silu(first[:, :64] + first[:, 64:]
                           + rad * wrv[:, :64] + ea * wev[:, :64])
        t_hi = jax.nn.silu(second[:, :64] + second[:, 64:]
                           + rad * wrv[:, 64:] + ea * wev[:, 64:])
        w2v = w2_ref[...]
        u = jax.nn.silu(jnp.dot(t_lo.astype(jnp.bfloat16), w2v[:64],
                                preferred_element_type=jnp.float32)
                        + jnp.dot(t_hi.astype(jnp.bfloat16), w2v[64:],
                                  preferred_element_type=jnp.float32)
                        + b2_ref[...])
        m = jnp.sum(u * w3_ref[...], axis=1, keepdims=True)
        lane = lax.broadcasted_iota(jnp.int32, d.shape, 1)
        t_ref[...] = jnp.where(lane < 3, d * m, 0.0)

    return pl.pallas_call(
        body,
        grid=(E // BLK_E,),
        in_specs=[pl.BlockSpec((BLK_E, H), lambda i: (i, 0)),
                  pl.BlockSpec((BLK_E, 16), lambda i: (i, 0)),
                  pl.BlockSpec((H, H), lambda i: (0, 0)),
                  pl.BlockSpec((1, H), lambda i: (0, 0)),
                  pl.BlockSpec((1, H), lambda i: (0, 0)),
                  pl.BlockSpec((1, H), lambda i: (0, 0)),
                  pl.BlockSpec((1, H), lambda i: (0, 0))],
        out_specs=pl.BlockSpec((BLK_E, 16), lambda i: (i, 0)),
        out_shape=jax.ShapeDtypeStruct((E, 16), jnp.float32),
    )(G, Dm, W2.astype(jnp.bfloat16), b2.reshape(1, H), w3.reshape(1, H),
      wr.reshape(1, H), we.reshape(1, H))


def _node(h, P0, P1, W3h, W3a, b3, W4, b4):
    N = h.shape[0]

    def body(h_ref, p0_ref, p1_ref, w3h_ref, w3a_ref, b3_ref, w4_ref, b4_ref, o_ref):
        hv = h_ref[...]
        agg = (p0_ref[...] + p1_ref[...]) * 0.01
        u = jax.nn.silu(jnp.dot(hv, w3h_ref[...], preferred_element_type=jnp.float32)
                        + jnp.dot(agg, w3a_ref[...], preferred_element_type=jnp.float32)
                        + b3_ref[...])
        o_ref[...] = hv + jnp.dot(u, w4_ref[...], preferred_element_type=jnp.float32) + b4_ref[...]

    return pl.pallas_call(
        body,
        grid=(N // BLK_N,),
        in_specs=[pl.BlockSpec((BLK_N, H), lambda i: (i, 0)),
                  pl.BlockSpec((BLK_N, H), lambda i: (i, 0)),
                  pl.BlockSpec((BLK_N, H), lambda i: (i, 0)),
                  pl.BlockSpec((H, H), lambda i: (0, 0)),
                  pl.BlockSpec((H, H), lambda i: (0, 0)),
                  pl.BlockSpec((1, H), lambda i: (0, 0)),
                  pl.BlockSpec((H, H), lambda i: (0, 0)),
                  pl.BlockSpec((1, H), lambda i: (0, 0))],
        out_specs=pl.BlockSpec((BLK_N, H), lambda i: (i, 0)),
        out_shape=jax.ShapeDtypeStruct((N, H), jnp.float32),
    )(h, P0, P1, W3h, W3a, b3.reshape(1, H), W4, b4.reshape(1, H))


def _xfinal(x, Q0, Q1):
    N = x.shape[0]

    def body(x_ref, q0_ref, q1_ref, o_ref):
        o_ref[...] = x_ref[...] + (q0_ref[...] + q1_ref[...])[:, :3] * 0.01

    return pl.pallas_call(
        body,
        grid=(N // BLK_N,),
        in_specs=[pl.BlockSpec((BLK_N, 3), lambda i: (i, 0)),
                  pl.BlockSpec((BLK_N, 16), lambda i: (i, 0)),
                  pl.BlockSpec((BLK_N, 16), lambda i: (i, 0))],
        out_specs=pl.BlockSpec((BLK_N, 3), lambda i: (i, 0)),
        out_shape=jax.ShapeDtypeStruct((N, 3), jnp.float32),
    )(x, Q0, Q1)


def kernel(h, x, edge_index, edge_attr, params):
    N = h.shape[0]
    row = edge_index[0].astype(jnp.int32)
    col = edge_index[1].astype(jnp.int32)

    E = row.shape[0]
    z128 = jnp.zeros((CH, H), jnp.float32)
    z16 = jnp.zeros((CH, 16), jnp.float32)

    xp = jnp.pad(x, ((0, 0), (0, 13)))
    XR, XC = _gather_pair(xp, xp, row, col)
    XR8 = XR.reshape(E // 8, 128)
    XC8 = XC.reshape(E // 8, 128)
    eap = jnp.pad(edge_attr, ((0, 0), (4, 11))).reshape(E // 8, 128)
    sg = jnp.kron(jnp.eye(8, dtype=jnp.float32), jnp.ones((16, 16), jnp.float32))
    Dm = _dprep(XR8, XC8, eap, sg).reshape(E, 16)

    hcur = h
    for p in params["gcl"]:
        W1 = p["W1"]
        A, B = _prep(hcur, W1[:H], W1[H:2 * H], p["b1"])
        G = _gather_fused(A, B, row, col)
        F = _edge_gcl(G, Dm, p["W2"], p["b2"], p["Wa"][:, 0], p["ba"],
                      W1[2 * H], W1[2 * H + 1])
        P = _sc_scatter_add(F, row, z128)
        hcur = _node(hcur, P[0, :N], P[1, :N], p["W3"][:H], p["W3"][H:],
                     p["b3"], p["W4"], p["b4"])

    c = params["coord"]
    W1 = c["W1"]
    A, B = _prep(hcur, W1[:H], W1[H:2 * H], c["b1"])
    G = _gather_fused(A, B, row, col)
    T = _edge_coord(G, Dm, c["W2"], c["b2"], c["W3"][:, 0],
                    W1[2 * H], W1[2 * H + 1])
    Q = _sc_scatter_add(T, row, z16)
    xout = _xfinal(x, Q[0, :N], Q[1, :N])
    return hcur, xout


# speedup vs baseline: 1.2714x; 1.2714x over previous
import functools

import jax
import jax.numpy as jnp
from jax import lax
from jax.experimental import pallas as pl
from jax.experimental.pallas import tpu as pltpu
from jax.experimental.pallas import tpu_sc as plsc

H = 128
CH = 128
NW = 32
NP = 10240
BLK_E = 1280
BLK_N = 2000
NSLICE = 2


def _sc_mesh():
    return plsc.VectorSubcoreMesh(core_axis_name="c", subcore_axis_name="s")


def _gather_pair(A, B, row, col):
    E = row.shape[0]
    D = A.shape[1]
    nch = E // CH
    per_w = (nch + NW - 1) // NW

    @functools.partial(
        pl.kernel,
        mesh=_sc_mesh(),
        compiler_params=pltpu.CompilerParams(use_tc_tiling_on_sc=(D % 128 == 0)),
        out_type=[jax.ShapeDtypeStruct((E, D), jnp.float32),
                  jax.ShapeDtypeStruct((E, D), jnp.float32)],
        scratch_types=[pltpu.VMEM((1, CH), jnp.int32),
                       pltpu.VMEM((1, CH), jnp.int32),
                       pltpu.VMEM((CH, D), jnp.float32),
                       pltpu.VMEM((CH, D), jnp.float32),
                       pltpu.SemaphoreType.DMA,
                       pltpu.SemaphoreType.DMA],
    )
    def k(a_hbm, b_hbm, row_hbm, col_hbm, ga_hbm, gb_hbm, ri, ci, ba, bb, s1, s2):
        wid = lax.axis_index("s") * 2 + lax.axis_index("c")

        @pl.loop(0, per_w)
        def _(i):
            c = wid + i * NW

            @pl.when(c < nch)
            def _():
                base = c * CH
                pltpu.sync_copy(row_hbm.at[pl.ds(base, CH)], ri.at[0])
                pltpu.sync_copy(col_hbm.at[pl.ds(base, CH)], ci.at[0])
                cp1 = pltpu.async_copy(a_hbm.at[ri.at[0]], ba, s1)
                cp2 = pltpu.async_copy(b_hbm.at[ci.at[0]], bb, s2)
                cp1.wait()
                cp2.wait()
                pltpu.sync_copy(ba, ga_hbm.at[pl.ds(base, CH)])
                pltpu.sync_copy(bb, gb_hbm.at[pl.ds(base, CH)])

    return k(A, B, row, col)


def _sc_scatter_add(F, row, zrows):
    E, D = F.shape
    nch = E // CH
    per_w = (nch + NW - 1) // NW
    rpt = NP // 16

    @functools.partial(
        pl.kernel,
        mesh=_sc_mesh(),
        compiler_params=pltpu.CompilerParams(use_tc_tiling_on_sc=(D % 128 == 0)),
        out_type=jax.ShapeDtypeStruct((2, NP, D), jnp.float32),
        scratch_types=[pltpu.VMEM((1, CH), jnp.int32),
                       pltpu.VMEM((CH, D), jnp.float32),
                       pltpu.VMEM_SHARED((NP, D), jnp.float32)],
    )
    def k(f_hbm, row_hbm, z_hbm, p_hbm, ri, fb, acc):
        cid = lax.axis_index("c")
        sid = lax.axis_index("s")
        wid = sid * 2 + cid

        @pl.loop(0, rpt // CH)
        def _(j):
            pltpu.sync_copy(z_hbm, acc.at[pl.ds(sid * rpt + j * CH, CH)])

        plsc.subcore_barrier()

        @pl.loop(0, per_w)
        def _(i):
            c = wid + i * NW

            @pl.when(c < nch)
            def _():
                base = c * CH
                pltpu.sync_copy(row_hbm.at[pl.ds(base, CH)], ri.at[0])
                pltpu.sync_copy(f_hbm.at[pl.ds(base, CH)], fb)
                pltpu.sync_copy(fb, acc.at[ri.at[0]], add=True)

        plsc.subcore_barrier()

        @pl.loop(0, rpt // CH)
        def _(j):
            off = sid * rpt + j * CH
            pltpu.sync_copy(acc.at[pl.ds(off, CH)], p_hbm.at[cid, pl.ds(off, CH)])

    return k(F, row, zrows)


def _prep(h, W1a, W1b, b1):
    N = h.shape[0]

    def body(h_ref, wa_ref, wb_ref, b_ref, a_ref, bo_ref):
        hv = h_ref[...]
        a_ref[...] = jnp.dot(hv, wa_ref[...], preferred_element_type=jnp.float32) + b_ref[...]
        bo_ref[...] = jnp.dot(hv, wb_ref[...], preferred_element_type=jnp.float32)

    return pl.pallas_call(
        body,
        grid=(N // BLK_N,),
        in_specs=[pl.BlockSpec((BLK_N, H), lambda i: (i, 0)),
                  pl.BlockSpec((H, H), lambda i: (0, 0)),
                  pl.BlockSpec((H, H), lambda i: (0, 0)),
                  pl.BlockSpec((1, H), lambda i: (0, 0))],
        out_specs=[pl.BlockSpec((BLK_N, H), lambda i: (i, 0)),
                   pl.BlockSpec((BLK_N, H), lambda i: (i, 0))],
        out_shape=[jax.ShapeDtypeStruct((N, H), jnp.float32),
                   jax.ShapeDtypeStruct((N, H), jnp.float32)],
    )(h, W1a, W1b, b1.reshape(1, H))


def _dprep(XR, XC, eattr):
    E = XR.shape[0]

    def body(xr_ref, xc_ref, ea_ref, d_ref):
        diff = xr_ref[...] - xc_ref[...]
        radial = jnp.sum(diff * diff, axis=1, keepdims=True)
        norm = jnp.sqrt(radial + 1e-8)
        cd = diff / (norm + 1.0)
        lane = lax.broadcasted_iota(jnp.int32, diff.shape, 1)
        d_ref[...] = jnp.where(lane < 3, cd,
                               jnp.where(lane == 3, radial,
                                         jnp.where(lane == 4, ea_ref[...], 0.0)))

    return pl.pallas_call(
        body,
        grid=(E // BLK_E,),
        in_specs=[pl.BlockSpec((BLK_E, 16), lambda i: (i, 0)),
                  pl.BlockSpec((BLK_E, 16), lambda i: (i, 0)),
                  pl.BlockSpec((BLK_E, 1), lambda i: (i, 0))],
        out_specs=pl.BlockSpec((BLK_E, 16), lambda i: (i, 0)),
        out_shape=jax.ShapeDtypeStruct((E, 16), jnp.float32),
    )(XR, XC, eattr)


def _edge_gcl(GA, GB, Dm, doff, W2, b2, wa, ba, wr, we):
    E = GA.shape[0]

    def body(ga_ref, gb_ref, d_ref, w2_ref, b2_ref, wa_ref, ba_ref, wr_ref, we_ref, f_ref):
        d = d_ref[...]
        pre = (ga_ref[...] + gb_ref[...]
               + d[:, 3:4] * wr_ref[...] + d[:, 4:5] * we_ref[...])
        t = jax.nn.silu(pre)
        mij = jax.nn.silu(jnp.dot(t, w2_ref[...], preferred_element_type=jnp.float32) + b2_ref[...])
        att = jax.nn.sigmoid(jnp.sum(mij * wa_ref[...], axis=1, keepdims=True) + ba_ref[:, 0:1])
        f_ref[...] = mij * att

    return pl.pallas_call(
        body,
        grid=(E // BLK_E,),
        in_specs=[pl.BlockSpec((BLK_E, H), lambda i: (i, 0)),
                  pl.BlockSpec((BLK_E, H), lambda i: (i, 0)),
                  pl.BlockSpec((BLK_E, 16), lambda i: (i + doff, 0)),
                  pl.BlockSpec((H, H), lambda i: (0, 0)),
                  pl.BlockSpec((1, H), lambda i: (0, 0)),
                  pl.BlockSpec((1, H), lambda i: (0, 0)),
                  pl.BlockSpec((1, H), lambda i: (0, 0)),
                  pl.BlockSpec((1, H), lambda i: (0, 0)),
                  pl.BlockSpec((1, H), lambda i: (0, 0))],
        out_specs=pl.BlockSpec((BLK_E, H), lambda i: (i, 0)),
        out_shape=jax.ShapeDtypeStruct((E, H), jnp.float32),
    )(GA, GB, Dm, W2, b2.reshape(1, H), wa.reshape(1, H),
      jnp.broadcast_to(ba.reshape(1, 1), (1, H)), wr.reshape(1, H), we.reshape(1, H))


def _edge_coord(GA, GB, Dm, doff, W2, b2, w3, wr, we):
    E = GA.shape[0]

    def body(ga_ref, gb_ref, d_ref, w2_ref, b2_ref, w3_ref, wr_ref, we_ref, t_ref):
        d = d_ref[...]
        pre = (ga_ref[...] + gb_ref[...]
               + d[:, 3:4] * wr_ref[...] + d[:, 4:5] * we_ref[...])
        t = jax.nn.silu(pre)
        u = jax.nn.silu(jnp.dot(t, w2_ref[...], preferred_element_type=jnp.float32) + b2_ref[...])
        m = jnp.sum(u * w3_ref[...], axis=1, keepdims=True)
        lane = lax.broadcasted_iota(jnp.int32, d.shape, 1)
        t_ref[...] = jnp.where(lane < 3, d * m, 0.0)

    return pl.pallas_call(
        body,
        grid=(E // BLK_E,),
        in_specs=[pl.BlockSpec((BLK_E, H), lambda i: (i, 0)),
                  pl.BlockSpec((BLK_E, H), lambda i: (i, 0)),
                  pl.BlockSpec((BLK_E, 16), lambda i: (i + doff, 0)),
                  pl.BlockSpec((H, H), lambda i: (0, 0)),
                  pl.BlockSpec((1, H), lambda i: (0, 0)),
                  pl.BlockSpec((1, H), lambda i: (0, 0)),
                  pl.BlockSpec((1, H), lambda i: (0, 0)),
                  pl.BlockSpec((1, H), lambda i: (0, 0))],
        out_specs=pl.BlockSpec((BLK_E, 16), lambda i: (i, 0)),
        out_shape=jax.ShapeDtypeStruct((E, 16), jnp.float32),
    )(GA, GB, Dm, W2, b2.reshape(1, H), w3.reshape(1, H), wr.reshape(1, H), we.reshape(1, H))


def _node(h, parts, W3h, W3a, b3, W4, b4):
    N = h.shape[0]
    npart = len(parts)

    def body(h_ref, *rest):
        p_refs = rest[:npart]
        w3h_ref, w3a_ref, b3_ref, w4_ref, b4_ref, o_ref = rest[npart:]
        hv = h_ref[...]
        agg = p_refs[0][...]
        for pr in p_refs[1:]:
            agg = agg + pr[...]
        agg = agg * 0.01
        u = jax.nn.silu(jnp.dot(hv, w3h_ref[...], preferred_element_type=jnp.float32)
                        + jnp.dot(agg, w3a_ref[...], preferred_element_type=jnp.float32)
                        + b3_ref[...])
        o_ref[...] = hv + jnp.dot(u, w4_ref[...], preferred_element_type=jnp.float32) + b4_ref[...]

    return pl.pallas_call(
        body,
        grid=(N // BLK_N,),
        in_specs=[pl.BlockSpec((BLK_N, H), lambda i: (i, 0))]
                 + [pl.BlockSpec((BLK_N, H), lambda i: (i, 0))] * npart
                 + [pl.BlockSpec((H, H), lambda i: (0, 0)),
                    pl.BlockSpec((H, H), lambda i: (0, 0)),
                    pl.BlockSpec((1, H), lambda i: (0, 0)),
                    pl.BlockSpec((H, H), lambda i: (0, 0)),
                    pl.BlockSpec((1, H), lambda i: (0, 0))],
        out_specs=pl.BlockSpec((BLK_N, H), lambda i: (i, 0)),
        out_shape=jax.ShapeDtypeStruct((N, H), jnp.float32),
    )(h, *parts, W3h, W3a, b3.reshape(1, H), W4, b4.reshape(1, H))


def _xfinal(x, parts):
    N = x.shape[0]
    npart = len(parts)

    def body(x_ref, *rest):
        q_refs = rest[:npart]
        o_ref = rest[npart]
        q = q_refs[0][...]
        for qr in q_refs[1:]:
            q = q + qr[...]
        o_ref[...] = x_ref[...] + q[:, :3] * 0.01

    return pl.pallas_call(
        body,
        grid=(N // BLK_N,),
        in_specs=[pl.BlockSpec((BLK_N, 3), lambda i: (i, 0))]
                 + [pl.BlockSpec((BLK_N, 16), lambda i: (i, 0))] * npart,
        out_specs=pl.BlockSpec((BLK_N, 3), lambda i: (i, 0)),
        out_shape=jax.ShapeDtypeStruct((N, 3), jnp.float32),
    )(x, *parts)


def kernel(h, x, edge_index, edge_attr, params):
    N = h.shape[0]
    row = edge_index[0].astype(jnp.int32)
    col = edge_index[1].astype(jnp.int32)
    E = row.shape[0]
    ES = E // NSLICE
    rows = [row[s * ES:(s + 1) * ES] for s in range(NSLICE)]
    cols = [col[s * ES:(s + 1) * ES] for s in range(NSLICE)]

    z128 = jnp.zeros((CH, H), jnp.float32)
    z16 = jnp.zeros((CH, 16), jnp.float32)

    xp = jnp.pad(x, ((0, 0), (0, 13)))
    XR, XC = _gather_pair(xp, xp, row, col)
    Dm = _dprep(XR, XC, edge_attr)

    hcur = h
    for p in params["gcl"]:
        W1 = p["W1"]
        A, B = _prep(hcur, W1[:H], W1[H:2 * H], p["b1"])
        Gs = [_gather_pair(A, B, rows[s], cols[s]) for s in range(NSLICE)]
        Fs = [_edge_gcl(Gs[s][0], Gs[s][1], Dm, s * (ES // BLK_E), p["W2"],
                        p["b2"], p["Wa"][:, 0], p["ba"], W1[2 * H], W1[2 * H + 1])
              for s in range(NSLICE)]
        Ps = [_sc_scatter_add(Fs[s], rows[s], z128) for s in range(NSLICE)]
        parts = [P[k, :N] for P in Ps for k in range(2)]
        hcur = _node(hcur, parts, p["W3"][:H], p["W3"][H:], p["b3"], p["W4"], p["b4"])

    c = params["coord"]
    W1 = c["W1"]
    A, B = _prep(hcur, W1[:H], W1[H:2 * H], c["b1"])
    Gs = [_gather_pair(A, B, rows[s], cols[s]) for s in range(NSLICE)]
    Ts = [_edge_coord(Gs[s][0], Gs[s][1], Dm, s * (ES // BLK_E), c["W2"],
                      c["b2"], c["W3"][:, 0], W1[2 * H], W1[2 * H + 1])
          for s in range(NSLICE)]
    Qs = [_sc_scatter_add(Ts[s], rows[s], z16) for s in range(NSLICE)]
    qparts = [Q[k, :N] for Q in Qs for k in range(2)]
    xout = _xfinal(x, qparts)
    return hcur, xout


# speedup vs baseline: 1.2910x; 1.0154x over previous
import functools

import jax
import jax.numpy as jnp
from jax import lax
from jax.experimental import pallas as pl
from jax.experimental.pallas import tpu as pltpu
from jax.experimental.pallas import tpu_sc as plsc

H = 128
CH = 128
NW = 32
NP = 10240
BLK_E = 1280
BLK_N = 2000
NSLICE = 4


def _sc_mesh():
    return plsc.VectorSubcoreMesh(core_axis_name="c", subcore_axis_name="s")


def _gather_pair(A, B, row, col):
    E = row.shape[0]
    D = A.shape[1]
    nch = E // CH
    per_w = (nch + NW - 1) // NW

    @functools.partial(
        pl.kernel,
        mesh=_sc_mesh(),
        compiler_params=pltpu.CompilerParams(use_tc_tiling_on_sc=(D % 128 == 0)),
        out_type=[jax.ShapeDtypeStruct((E, D), jnp.float32),
                  jax.ShapeDtypeStruct((E, D), jnp.float32)],
        scratch_types=[pltpu.VMEM((1, CH), jnp.int32),
                       pltpu.VMEM((1, CH), jnp.int32),
                       pltpu.VMEM((CH, D), jnp.float32),
                       pltpu.VMEM((CH, D), jnp.float32),
                       pltpu.SemaphoreType.DMA,
                       pltpu.SemaphoreType.DMA],
    )
    def k(a_hbm, b_hbm, row_hbm, col_hbm, ga_hbm, gb_hbm, ri, ci, ba, bb, s1, s2):
        wid = lax.axis_index("s") * 2 + lax.axis_index("c")

        @pl.loop(0, per_w)
        def _(i):
            c = wid + i * NW

            @pl.when(c < nch)
            def _():
                base = c * CH
                pltpu.sync_copy(row_hbm.at[pl.ds(base, CH)], ri.at[0])
                pltpu.sync_copy(col_hbm.at[pl.ds(base, CH)], ci.at[0])
                cp1 = pltpu.async_copy(a_hbm.at[ri.at[0]], ba, s1)
                cp2 = pltpu.async_copy(b_hbm.at[ci.at[0]], bb, s2)
                cp1.wait()
                cp2.wait()
                pltpu.sync_copy(ba, ga_hbm.at[pl.ds(base, CH)])
                pltpu.sync_copy(bb, gb_hbm.at[pl.ds(base, CH)])

    return k(A, B, row, col)


def _sc_scatter_add(F, row, zrows):
    E, D = F.shape
    nch = E // CH
    per_w = (nch + NW - 1) // NW
    rpt = NP // 16

    @functools.partial(
        pl.kernel,
        mesh=_sc_mesh(),
        compiler_params=pltpu.CompilerParams(use_tc_tiling_on_sc=(D % 128 == 0)),
        out_type=jax.ShapeDtypeStruct((2, NP, D), jnp.float32),
        scratch_types=[pltpu.VMEM((1, CH), jnp.int32),
                       pltpu.VMEM((CH, D), jnp.float32),
                       pltpu.VMEM_SHARED((NP, D), jnp.float32)],
    )
    def k(f_hbm, row_hbm, z_hbm, p_hbm, ri, fb, acc):
        cid = lax.axis_index("c")
        sid = lax.axis_index("s")
        wid = sid * 2 + cid

        @pl.loop(0, rpt // CH)
        def _(j):
            pltpu.sync_copy(z_hbm, acc.at[pl.ds(sid * rpt + j * CH, CH)])

        plsc.subcore_barrier()

        @pl.loop(0, per_w)
        def _(i):
            c = wid + i * NW

            @pl.when(c < nch)
            def _():
                base = c * CH
                pltpu.sync_copy(row_hbm.at[pl.ds(base, CH)], ri.at[0])
                pltpu.sync_copy(f_hbm.at[pl.ds(base, CH)], fb)
                pltpu.sync_copy(fb, acc.at[ri.at[0]], add=True)

        plsc.subcore_barrier()

        @pl.loop(0, rpt // CH)
        def _(j):
            off = sid * rpt + j * CH
            pltpu.sync_copy(acc.at[pl.ds(off, CH)], p_hbm.at[cid, pl.ds(off, CH)])

    return k(F, row, zrows)


def _prep(h, W1a, W1b, b1):
    N = h.shape[0]

    def body(h_ref, wa_ref, wb_ref, b_ref, a_ref, bo_ref):
        hv = h_ref[...]
        a_ref[...] = jnp.dot(hv, wa_ref[...], preferred_element_type=jnp.float32) + b_ref[...]
        bo_ref[...] = jnp.dot(hv, wb_ref[...], preferred_element_type=jnp.float32)

    return pl.pallas_call(
        body,
        grid=(N // BLK_N,),
        in_specs=[pl.BlockSpec((BLK_N, H), lambda i: (i, 0)),
                  pl.BlockSpec((H, H), lambda i: (0, 0)),
                  pl.BlockSpec((H, H), lambda i: (0, 0)),
                  pl.BlockSpec((1, H), lambda i: (0, 0))],
        out_specs=[pl.BlockSpec((BLK_N, H), lambda i: (i, 0)),
                   pl.BlockSpec((BLK_N, H), lambda i: (i, 0))],
        out_shape=[jax.ShapeDtypeStruct((N, H), jnp.float32),
                   jax.ShapeDtypeStruct((N, H), jnp.float32)],
    )(h, W1a, W1b, b1.reshape(1, H))


def _dprep(XR, XC, eattr):
    E = XR.shape[0]

    def body(xr_ref, xc_ref, ea_ref, d_ref):
        diff = xr_ref[...] - xc_ref[...]
        radial = jnp.sum(diff * diff, axis=1, keepdims=True)
        norm = jnp.sqrt(radial + 1e-8)
        cd = diff / (norm + 1.0)
        lane = lax.broadcasted_iota(jnp.int32, diff.shape, 1)
        d_ref[...] = jnp.where(lane < 3, cd,
                               jnp.where(lane == 3, radial,
                                         jnp.where(lane == 4, ea_ref[...], 0.0)))

    return pl.pallas_call(
        body,
        grid=(E // BLK_E,),
        in_specs=[pl.BlockSpec((BLK_E, 16), lambda i: (i, 0)),
                  pl.BlockSpec((BLK_E, 16), lambda i: (i, 0)),
                  pl.BlockSpec((BLK_E, 1), lambda i: (i, 0))],
        out_specs=pl.BlockSpec((BLK_E, 16), lambda i: (i, 0)),
        out_shape=jax.ShapeDtypeStruct((E, 16), jnp.float32),
    )(XR, XC, eattr)


def _edge_gcl(GA, GB, Dm, doff, W2, b2, wa, ba, wr, we):
    E = GA.shape[0]

    def body(ga_ref, gb_ref, d_ref, w2_ref, b2_ref, wa_ref, ba_ref, wr_ref, we_ref, f_ref):
        d = d_ref[...]
        pre = (ga_ref[...] + gb_ref[...]
               + d[:, 3:4] * wr_ref[...] + d[:, 4:5] * we_ref[...])
        t = jax.nn.silu(pre)
        mij = jax.nn.silu(jnp.dot(t, w2_ref[...], preferred_element_type=jnp.float32) + b2_ref[...])
        att = jax.nn.sigmoid(jnp.sum(mij * wa_ref[...], axis=1, keepdims=True) + ba_ref[:, 0:1])
        f_ref[...] = mij * att

    return pl.pallas_call(
        body,
        grid=(E // BLK_E,),
        in_specs=[pl.BlockSpec((BLK_E, H), lambda i: (i, 0)),
                  pl.BlockSpec((BLK_E, H), lambda i: (i, 0)),
                  pl.BlockSpec((BLK_E, 16), lambda i: (i + doff, 0)),
                  pl.BlockSpec((H, H), lambda i: (0, 0)),
                  pl.BlockSpec((1, H), lambda i: (0, 0)),
                  pl.BlockSpec((1, H), lambda i: (0, 0)),
                  pl.BlockSpec((1, H), lambda i: (0, 0)),
                  pl.BlockSpec((1, H), lambda i: (0, 0)),
                  pl.BlockSpec((1, H), lambda i: (0, 0))],
        out_specs=pl.BlockSpec((BLK_E, H), lambda i: (i, 0)),
        out_shape=jax.ShapeDtypeStruct((E, H), jnp.float32),
    )(GA, GB, Dm, W2, b2.reshape(1, H), wa.reshape(1, H),
      jnp.broadcast_to(ba.reshape(1, 1), (1, H)), wr.reshape(1, H), we.reshape(1, H))


def _edge_coord(GA, GB, Dm, doff, W2, b2, w3, wr, we):
    E = GA.shape[0]

    def body(ga_ref, gb_ref, d_ref, w2_ref, b2_ref, w3_ref, wr_ref, we_ref, t_ref):
        d = d_ref[...]
        pre = (ga_ref[...] + gb_ref[...]
               + d[:, 3:4] * wr_ref[...] + d[:, 4:5] * we_ref[...])
        t = jax.nn.silu(pre)
        u = jax.nn.silu(jnp.dot(t, w2_ref[...], preferred_element_type=jnp.float32) + b2_ref[...])
        m = jnp.sum(u * w3_ref[...], axis=1, keepdims=True)
        lane = lax.broadcasted_iota(jnp.int32, d.shape, 1)
        t_ref[...] = jnp.where(lane < 3, d * m, 0.0)

    return pl.pallas_call(
        body,
        grid=(E // BLK_E,),
        in_specs=[pl.BlockSpec((BLK_E, H), lambda i: (i, 0)),
                  pl.BlockSpec((BLK_E, H), lambda i: (i, 0)),
                  pl.BlockSpec((BLK_E, 16), lambda i: (i + doff, 0)),
                  pl.BlockSpec((H, H), lambda i: (0, 0)),
                  pl.BlockSpec((1, H), lambda i: (0, 0)),
                  pl.BlockSpec((1, H), lambda i: (0, 0)),
                  pl.BlockSpec((1, H), lambda i: (0, 0)),
                  pl.BlockSpec((1, H), lambda i: (0, 0))],
        out_specs=pl.BlockSpec((BLK_E, 16), lambda i: (i, 0)),
        out_shape=jax.ShapeDtypeStruct((E, 16), jnp.float32),
    )(GA, GB, Dm, W2, b2.reshape(1, H), w3.reshape(1, H), wr.reshape(1, H), we.reshape(1, H))


def _node(h, parts, W3h, W3a, b3, W4, b4):
    N = h.shape[0]
    npart = len(parts)

    def body(h_ref, *rest):
        p_refs = rest[:npart]
        w3h_ref, w3a_ref, b3_ref, w4_ref, b4_ref, o_ref = rest[npart:]
        hv = h_ref[...]
        agg = p_refs[0][...]
        for pr in p_refs[1:]:
            agg = agg + pr[...]
        agg = agg * 0.01
        u = jax.nn.silu(jnp.dot(hv, w3h_ref[...], preferred_element_type=jnp.float32)
                        + jnp.dot(agg, w3a_ref[...], preferred_element_type=jnp.float32)
                        + b3_ref[...])
        o_ref[...] = hv + jnp.dot(u, w4_ref[...], preferred_element_type=jnp.float32) + b4_ref[...]

    return pl.pallas_call(
        body,
        grid=(N // BLK_N,),
        in_specs=[pl.BlockSpec((BLK_N, H), lambda i: (i, 0))]
                 + [pl.BlockSpec((BLK_N, H), lambda i: (i, 0))] * npart
                 + [pl.BlockSpec((H, H), lambda i: (0, 0)),
                    pl.BlockSpec((H, H), lambda i: (0, 0)),
                    pl.BlockSpec((1, H), lambda i: (0, 0)),
                    pl.BlockSpec((H, H), lambda i: (0, 0)),
                    pl.BlockSpec((1, H), lambda i: (0, 0))],
        out_specs=pl.BlockSpec((BLK_N, H), lambda i: (i, 0)),
        out_shape=jax.ShapeDtypeStruct((N, H), jnp.float32),
    )(h, *parts, W3h, W3a, b3.reshape(1, H), W4, b4.reshape(1, H))


def _xfinal(x, parts):
    N = x.shape[0]
    npart = len(parts)

    def body(x_ref, *rest):
        q_refs = rest[:npart]
        o_ref = rest[npart]
        q = q_refs[0][...]
        for qr in q_refs[1:]:
            q = q + qr[...]
        o_ref[...] = x_ref[...] + q[:, :3] * 0.01

    return pl.pallas_call(
        body,
        grid=(N // BLK_N,),
        in_specs=[pl.BlockSpec((BLK_N, 3), lambda i: (i, 0))]
                 + [pl.BlockSpec((BLK_N, 16), lambda i: (i, 0))] * npart,
        out_specs=pl.BlockSpec((BLK_N, 3), lambda i: (i, 0)),
        out_shape=jax.ShapeDtypeStruct((N, 3), jnp.float32),
    )(x, *parts)


def kernel(h, x, edge_index, edge_attr, params):
    N = h.shape[0]
    row = edge_index[0].astype(jnp.int32)
    col = edge_index[1].astype(jnp.int32)
    E = row.shape[0]
    ES = E // NSLICE
    rows = [row[s * ES:(s + 1) * ES] for s in range(NSLICE)]
    cols = [col[s * ES:(s + 1) * ES] for s in range(NSLICE)]

    z128 = jnp.zeros((CH, H), jnp.float32)
    z16 = jnp.zeros((CH, 16), jnp.float32)

    xp = jnp.pad(x, ((0, 0), (0, 13)))
    XR, XC = _gather_pair(xp, xp, row, col)
    Dm = _dprep(XR, XC, edge_attr)

    hcur = h
    for p in params["gcl"]:
        W1 = p["W1"]
        A, B = _prep(hcur, W1[:H], W1[H:2 * H], p["b1"])
        Gs = [_gather_pair(A, B, rows[s], cols[s]) for s in range(NSLICE)]
        Fs = [_edge_gcl(Gs[s][0], Gs[s][1], Dm, s * (ES // BLK_E), p["W2"],
                        p["b2"], p["Wa"][:, 0], p["ba"], W1[2 * H], W1[2 * H + 1])
              for s in range(NSLICE)]
        Ps = [_sc_scatter_add(Fs[s], rows[s], z128) for s in range(NSLICE)]
        parts = [P[k, :N] for P in Ps for k in range(2)]
        hcur = _node(hcur, parts, p["W3"][:H], p["W3"][H:], p["b3"], p["W4"], p["b4"])

    c = params["coord"]
    W1 = c["W1"]
    A, B = _prep(hcur, W1[:H], W1[H:2 * H], c["b1"])
    Gs = [_gather_pair(A, B, rows[s], cols[s]) for s in range(NSLICE)]
    Ts = [_edge_coord(Gs[s][0], Gs[s][1], Dm, s * (ES // BLK_E), c["W2"],
                      c["b2"], c["W3"][:, 0], W1[2 * H], W1[2 * H + 1])
          for s in range(NSLICE)]
    Qs = [_sc_scatter_add(Ts[s], rows[s], z16) for s in range(NSLICE)]
    qparts = [Q[k, :N] for Q in Qs for k in range(2)]
    xout = _xfinal(x, qparts)
    return hcur, xout


# speedup vs baseline: 1.3200x; 1.0224x over previous
import functools

import jax
import jax.numpy as jnp
from jax import lax
from jax.experimental import pallas as pl
from jax.experimental.pallas import tpu as pltpu
from jax.experimental.pallas import tpu_sc as plsc

H = 128
CH = 128
NW = 32
NP = 10240
BLK_E = 1600
BLK_N = 2000
NSLICE = 4


def _sc_mesh():
    return plsc.VectorSubcoreMesh(core_axis_name="c", subcore_axis_name="s")


def _gather_pair(A, B, row, col):
    E = row.shape[0]
    D = A.shape[1]
    nch = E // CH
    per_w = (nch + NW - 1) // NW

    @functools.partial(
        pl.kernel,
        mesh=_sc_mesh(),
        compiler_params=pltpu.CompilerParams(use_tc_tiling_on_sc=(D % 128 == 0)),
        out_type=[jax.ShapeDtypeStruct((E, D), jnp.float32),
                  jax.ShapeDtypeStruct((E, D), jnp.float32)],
        scratch_types=[pltpu.VMEM((1, CH), jnp.int32),
                       pltpu.VMEM((1, CH), jnp.int32),
                       pltpu.VMEM((CH, D), jnp.float32),
                       pltpu.VMEM((CH, D), jnp.float32),
                       pltpu.SemaphoreType.DMA,
                       pltpu.SemaphoreType.DMA],
    )
    def k(a_hbm, b_hbm, row_hbm, col_hbm, ga_hbm, gb_hbm, ri, ci, ba, bb, s1, s2):
        wid = lax.axis_index("s") * 2 + lax.axis_index("c")

        @pl.loop(0, per_w)
        def _(i):
            c = wid + i * NW

            @pl.when(c < nch)
            def _():
                base = c * CH
                pltpu.sync_copy(row_hbm.at[pl.ds(base, CH)], ri.at[0])
                pltpu.sync_copy(col_hbm.at[pl.ds(base, CH)], ci.at[0])
                cp1 = pltpu.async_copy(a_hbm.at[ri.at[0]], ba, s1)
                cp2 = pltpu.async_copy(b_hbm.at[ci.at[0]], bb, s2)
                cp1.wait()
                cp2.wait()
                pltpu.sync_copy(ba, ga_hbm.at[pl.ds(base, CH)])
                pltpu.sync_copy(bb, gb_hbm.at[pl.ds(base, CH)])

    return k(A, B, row, col)


def _sc_scatter_add(F, row, zrows):
    E, D = F.shape
    nch = E // CH
    per_w = (nch + NW - 1) // NW
    rpt = NP // 16

    @functools.partial(
        pl.kernel,
        mesh=_sc_mesh(),
        compiler_params=pltpu.CompilerParams(use_tc_tiling_on_sc=(D % 128 == 0)),
        out_type=jax.ShapeDtypeStruct((2, NP, D), jnp.float32),
        scratch_types=[pltpu.VMEM((1, CH), jnp.int32),
                       pltpu.VMEM((CH, D), jnp.float32),
                       pltpu.VMEM_SHARED((NP, D), jnp.float32)],
    )
    def k(f_hbm, row_hbm, z_hbm, p_hbm, ri, fb, acc):
        cid = lax.axis_index("c")
        sid = lax.axis_index("s")
        wid = sid * 2 + cid

        @pl.loop(0, rpt // CH)
        def _(j):
            pltpu.sync_copy(z_hbm, acc.at[pl.ds(sid * rpt + j * CH, CH)])

        plsc.subcore_barrier()

        @pl.loop(0, per_w)
        def _(i):
            c = wid + i * NW

            @pl.when(c < nch)
            def _():
                base = c * CH
                pltpu.sync_copy(row_hbm.at[pl.ds(base, CH)], ri.at[0])
                pltpu.sync_copy(f_hbm.at[pl.ds(base, CH)], fb)
                pltpu.sync_copy(fb, acc.at[ri.at[0]], add=True)

        plsc.subcore_barrier()

        @pl.loop(0, rpt // CH)
        def _(j):
            off = sid * rpt + j * CH
            pltpu.sync_copy(acc.at[pl.ds(off, CH)], p_hbm.at[cid, pl.ds(off, CH)])

    return k(F, row, zrows)


def _prep(h, W1a, W1b, b1):
    N = h.shape[0]

    def body(h_ref, wa_ref, wb_ref, b_ref, a_ref, bo_ref):
        hv = h_ref[...]
        a_ref[...] = jnp.dot(hv, wa_ref[...], preferred_element_type=jnp.float32) + b_ref[...]
        bo_ref[...] = jnp.dot(hv, wb_ref[...], preferred_element_type=jnp.float32)

    return pl.pallas_call(
        body,
        grid=(N // BLK_N,),
        in_specs=[pl.BlockSpec((BLK_N, H), lambda i: (i, 0)),
                  pl.BlockSpec((H, H), lambda i: (0, 0)),
                  pl.BlockSpec((H, H), lambda i: (0, 0)),
                  pl.BlockSpec((1, H), lambda i: (0, 0))],
        out_specs=[pl.BlockSpec((BLK_N, H), lambda i: (i, 0)),
                   pl.BlockSpec((BLK_N, H), lambda i: (i, 0))],
        out_shape=[jax.ShapeDtypeStruct((N, H), jnp.float32),
                   jax.ShapeDtypeStruct((N, H), jnp.float32)],
    )(h, W1a, W1b, b1.reshape(1, H))


def _dprep(XR, XC, eattr):
    E = XR.shape[0]

    def body(xr_ref, xc_ref, ea_ref, d_ref):
        diff = xr_ref[...] - xc_ref[...]
        radial = jnp.sum(diff * diff, axis=1, keepdims=True)
        norm = jnp.sqrt(radial + 1e-8)
        cd = diff / (norm + 1.0)
        lane = lax.broadcasted_iota(jnp.int32, diff.shape, 1)
        d_ref[...] = jnp.where(lane < 3, cd,
                               jnp.where(lane == 3, radial,
                                         jnp.where(lane == 4, ea_ref[...], 0.0)))

    return pl.pallas_call(
        body,
        grid=(E // BLK_E,),
        in_specs=[pl.BlockSpec((BLK_E, 16), lambda i: (i, 0)),
                  pl.BlockSpec((BLK_E, 16), lambda i: (i, 0)),
                  pl.BlockSpec((BLK_E, 1), lambda i: (i, 0))],
        out_specs=pl.BlockSpec((BLK_E, 16), lambda i: (i, 0)),
        out_shape=jax.ShapeDtypeStruct((E, 16), jnp.float32),
    )(XR, XC, eattr)


def _edge_gcl(GA, GB, Dm, doff, W2, b2, wa, ba, wr, we):
    E = GA.shape[0]

    def body(ga_ref, gb_ref, d_ref, w2_ref, b2_ref, wa_ref, ba_ref, wr_ref, we_ref, f_ref):
        d = d_ref[...]
        pre = (ga_ref[...] + gb_ref[...]
               + d[:, 3:4] * wr_ref[...] + d[:, 4:5] * we_ref[...])
        t = jax.nn.silu(pre)
        mij = jax.nn.silu(jnp.dot(t, w2_ref[...], preferred_element_type=jnp.float32) + b2_ref[...])
        att = jax.nn.sigmoid(jnp.sum(mij * wa_ref[...], axis=1, keepdims=True) + ba_ref[:, 0:1])
        f_ref[...] = mij * att

    return pl.pallas_call(
        body,
        grid=(E // BLK_E,),
        in_specs=[pl.BlockSpec((BLK_E, H), lambda i: (i, 0)),
                  pl.BlockSpec((BLK_E, H), lambda i: (i, 0)),
                  pl.BlockSpec((BLK_E, 16), lambda i: (i + doff, 0)),
                  pl.BlockSpec((H, H), lambda i: (0, 0)),
                  pl.BlockSpec((1, H), lambda i: (0, 0)),
                  pl.BlockSpec((1, H), lambda i: (0, 0)),
                  pl.BlockSpec((1, H), lambda i: (0, 0)),
                  pl.BlockSpec((1, H), lambda i: (0, 0)),
                  pl.BlockSpec((1, H), lambda i: (0, 0))],
        out_specs=pl.BlockSpec((BLK_E, H), lambda i: (i, 0)),
        out_shape=jax.ShapeDtypeStruct((E, H), jnp.float32),
    )(GA, GB, Dm, W2, b2.reshape(1, H), wa.reshape(1, H),
      jnp.broadcast_to(ba.reshape(1, 1), (1, H)), wr.reshape(1, H), we.reshape(1, H))


def _edge_coord(GA, GB, Dm, doff, W2, b2, w3, wr, we):
    E = GA.shape[0]

    def body(ga_ref, gb_ref, d_ref, w2_ref, b2_ref, w3_ref, wr_ref, we_ref, t_ref):
        d = d_ref[...]
        pre = (ga_ref[...] + gb_ref[...]
               + d[:, 3:4] * wr_ref[...] + d[:, 4:5] * we_ref[...])
        t = jax.nn.silu(pre)
        u = jax.nn.silu(jnp.dot(t, w2_ref[...], preferred_element_type=jnp.float32) + b2_ref[...])
        m = jnp.sum(u * w3_ref[...], axis=1, keepdims=True)
        lane = lax.broadcasted_iota(jnp.int32, d.shape, 1)
        t_ref[...] = jnp.where(lane < 3, d * m, 0.0)

    return pl.pallas_call(
        body,
        grid=(E // BLK_E,),
        in_specs=[pl.BlockSpec((BLK_E, H), lambda i: (i, 0)),
                  pl.BlockSpec((BLK_E, H), lambda i: (i, 0)),
                  pl.BlockSpec((BLK_E, 16), lambda i: (i + doff, 0)),
                  pl.BlockSpec((H, H), lambda i: (0, 0)),
                  pl.BlockSpec((1, H), lambda i: (0, 0)),
                  pl.BlockSpec((1, H), lambda i: (0, 0)),
                  pl.BlockSpec((1, H), lambda i: (0, 0)),
                  pl.BlockSpec((1, H), lambda i: (0, 0))],
        out_specs=pl.BlockSpec((BLK_E, 16), lambda i: (i, 0)),
        out_shape=jax.ShapeDtypeStruct((E, 16), jnp.float32),
    )(GA, GB, Dm, W2, b2.reshape(1, H), w3.reshape(1, H), wr.reshape(1, H), we.reshape(1, H))


def _node(h, parts, W3h, W3a, b3, W4, b4):
    N = h.shape[0]
    npart = len(parts)

    def body(h_ref, *rest):
        p_refs = rest[:npart]
        w3h_ref, w3a_ref, b3_ref, w4_ref, b4_ref, o_ref = rest[npart:]
        hv = h_ref[...]
        agg = p_refs[0][...]
        for pr in p_refs[1:]:
            agg = agg + pr[...]
        agg = agg * 0.01
        u = jax.nn.silu(jnp.dot(hv, w3h_ref[...], preferred_element_type=jnp.float32)
                        + jnp.dot(agg, w3a_ref[...], preferred_element_type=jnp.float32)
                        + b3_ref[...])
        o_ref[...] = hv + jnp.dot(u, w4_ref[...], preferred_element_type=jnp.float32) + b4_ref[...]

    return pl.pallas_call(
        body,
        grid=(N // BLK_N,),
        in_specs=[pl.BlockSpec((BLK_N, H), lambda i: (i, 0))]
                 + [pl.BlockSpec((BLK_N, H), lambda i: (i, 0))] * npart
                 + [pl.BlockSpec((H, H), lambda i: (0, 0)),
                    pl.BlockSpec((H, H), lambda i: (0, 0)),
                    pl.BlockSpec((1, H), lambda i: (0, 0)),
                    pl.BlockSpec((H, H), lambda i: (0, 0)),
                    pl.BlockSpec((1, H), lambda i: (0, 0))],
        out_specs=pl.BlockSpec((BLK_N, H), lambda i: (i, 0)),
        out_shape=jax.ShapeDtypeStruct((N, H), jnp.float32),
    )(h, *parts, W3h, W3a, b3.reshape(1, H), W4, b4.reshape(1, H))


def _xfinal(x, parts):
    N = x.shape[0]
    npart = len(parts)

    def body(x_ref, *rest):
        q_refs = rest[:npart]
        o_ref = rest[npart]
        q = q_refs[0][...]
        for qr in q_refs[1:]:
            q = q + qr[...]
        o_ref[...] = x_ref[...] + q[:, :3] * 0.01

    return pl.pallas_call(
        body,
        grid=(N // BLK_N,),
        in_specs=[pl.BlockSpec((BLK_N, 3), lambda i: (i, 0))]
                 + [pl.BlockSpec((BLK_N, 16), lambda i: (i, 0))] * npart,
        out_specs=pl.BlockSpec((BLK_N, 3), lambda i: (i, 0)),
        out_shape=jax.ShapeDtypeStruct((N, 3), jnp.float32),
    )(x, *parts)


def kernel(h, x, edge_index, edge_attr, params):
    N = h.shape[0]
    row = edge_index[0].astype(jnp.int32)
    col = edge_index[1].astype(jnp.int32)
    E = row.shape[0]
    ES = E // NSLICE
    rows = [row[s * ES:(s + 1) * ES] for s in range(NSLICE)]
    cols = [col[s * ES:(s + 1) * ES] for s in range(NSLICE)]

    z128 = jnp.zeros((CH, H), jnp.float32)
    z16 = jnp.zeros((CH, 16), jnp.float32)

    xp = jnp.pad(x, ((0, 0), (0, 13)))
    XR, XC = _gather_pair(xp, xp, row, col)
    Dm = _dprep(XR, XC, edge_attr)

    hcur = h
    for p in params["gcl"]:
        W1 = p["W1"]
        A, B = _prep(hcur, W1[:H], W1[H:2 * H], p["b1"])
        Gs = [_gather_pair(A, B, rows[s], cols[s]) for s in range(NSLICE)]
        Fs = [_edge_gcl(Gs[s][0], Gs[s][1], Dm, s * (ES // BLK_E), p["W2"],
                        p["b2"], p["Wa"][:, 0], p["ba"], W1[2 * H], W1[2 * H + 1])
              for s in range(NSLICE)]
        Ps = [_sc_scatter_add(Fs[s], rows[s], z128) for s in range(NSLICE)]
        parts = [P[k, :N] for P in Ps for k in range(2)]
        hcur = _node(hcur, parts, p["W3"][:H], p["W3"][H:], p["b3"], p["W4"], p["b4"])

    c = params["coord"]
    W1 = c["W1"]
    A, B = _prep(hcur, W1[:H], W1[H:2 * H], c["b1"])
    Gs = [_gather_pair(A, B, rows[s], cols[s]) for s in range(NSLICE)]
    Ts = [_edge_coord(Gs[s][0], Gs[s][1], Dm, s * (ES // BLK_E), c["W2"],
                      c["b2"], c["W3"][:, 0], W1[2 * H], W1[2 * H + 1])
          for s in range(NSLICE)]
    Qs = [_sc_scatter_add(Ts[s], rows[s], z16) for s in range(NSLICE)]
    qparts = [Q[k, :N] for Q in Qs for k in range(2)]
    xout = _xfinal(x, qparts)
    return hcur, xout
